# baseline (device time: 51023 ns/iter reference)
import jax
import jax.numpy as jnp
from jax import lax
from jax.experimental import pallas as pl
from jax.experimental.pallas import tpu as pltpu

N_DEV = 4
M = 1024
N = 1024
H = M // 2
Q = M // 4


def kernel(A, B):
    def body(
        a_ref,
        b_ref,
        out_ref,
        acc_ref,
        h_send,
        h_recv,
        q_send,
        q_recv,
        gq_send,
        gq_recv,
        gh_send,
        gh_recv,
        send_sems,
        recv_sems,
    ):
        d = lax.axis_index("i")
        p1 = d ^ 1
        p2 = 3 - d
        keep_half = (d ^ (d >> 1)) & 1
        send_half = 1 - keep_half
        keep_r = keep_half * H
        send_r = send_half * H
        qi = keep_half * 2 + (d >> 1)
        qo = keep_half * 2 + (1 - (d >> 1))
        my_q_r = qi * Q
        other_q_r = qo * Q

        barrier_sem = pltpu.get_barrier_semaphore()
        for nbr in [p1, p2]:
            pl.semaphore_signal(
                barrier_sem,
                inc=1,
                device_id=(nbr,),
                device_id_type=pl.DeviceIdType.MESH,
            )
        pl.semaphore_wait(barrier_sem, 2)

        a = a_ref[...].astype(jnp.bfloat16)
        b = b_ref[...].astype(jnp.bfloat16)
        acc_ref[...] = jnp.dot(a, b, preferred_element_type=jnp.float32)

        h_send[...] = acc_ref[pl.ds(send_r, H), :].astype(jnp.bfloat16)
        rdma1 = pltpu.make_async_remote_copy(
            src_ref=h_send,
            dst_ref=h_recv,
            send_sem=send_sems.at[0],
            recv_sem=recv_sems.at[0],
            device_id=(p1,),
            device_id_type=pl.DeviceIdType.MESH,
        )
        rdma1.start()
        rdma1.wait()
        acc_ref[pl.ds(keep_r, H), :] = (
            acc_ref[pl.ds(keep_r, H), :] + h_recv[...].astype(jnp.float32)
        )

        q_send[...] = acc_ref[pl.ds(other_q_r, Q), :].astype(jnp.bfloat16)
        rdma2 = pltpu.make_async_remote_copy(
            src_ref=q_send,
            dst_ref=q_recv,
            send_sem=send_sems.at[1],
            recv_sem=recv_sems.at[1],
            device_id=(p2,),
            device_id_type=pl.DeviceIdType.MESH,
        )
        rdma2.start()
        rdma2.wait()
        zq = acc_ref[pl.ds(my_q_r, Q), :] + q_recv[...].astype(jnp.float32)

        g = 0.5 * zq * (
            1.0 + jnp.tanh(0.7978845608 * (zq + 0.044715 * zq * zq * zq))
        )
        out_ref[pl.ds(my_q_r, Q), :] = g

        gq_send[...] = g.astype(jnp.bfloat16)
        rdma3 = pltpu.make_async_remote_copy(
            src_ref=gq_send,
            dst_ref=gq_recv,
            send_sem=send_sems.at[2],
            recv_sem=recv_sems.at[2],
            device_id=(p2,),
            device_id_type=pl.DeviceIdType.MESH,
        )
        rdma3.start()
        rdma3.wait()
        out_ref[pl.ds(other_q_r, Q), :] = gq_recv[...].astype(jnp.float32)

        gh_send[...] = out_ref[pl.ds(keep_r, H), :].astype(jnp.bfloat16)
        rdma4 = pltpu.make_async_remote_copy(
            src_ref=gh_send,
            dst_ref=gh_recv,
            send_sem=send_sems.at[3],
            recv_sem=recv_sems.at[3],
            device_id=(p1,),
            device_id_type=pl.DeviceIdType.MESH,
        )
        rdma4.start()
        rdma4.wait()
        out_ref[pl.ds(send_r, H), :] = gh_recv[...].astype(jnp.float32)

    return pl.pallas_call(
        body,
        out_shape=jax.ShapeDtypeStruct((M, N), jnp.float32),
        in_specs=[
            pl.BlockSpec(memory_space=pltpu.VMEM),
            pl.BlockSpec(memory_space=pltpu.VMEM),
        ],
        out_specs=pl.BlockSpec(memory_space=pltpu.VMEM),
        scratch_shapes=[
            pltpu.VMEM((M, N), jnp.float32),
            pltpu.VMEM((H, N), jnp.bfloat16),
            pltpu.VMEM((H, N), jnp.bfloat16),
            pltpu.VMEM((Q, N), jnp.bfloat16),
            pltpu.VMEM((Q, N), jnp.bfloat16),
            pltpu.VMEM((Q, N), jnp.bfloat16),
            pltpu.VMEM((Q, N), jnp.bfloat16),
            pltpu.VMEM((H, N), jnp.bfloat16),
            pltpu.VMEM((H, N), jnp.bfloat16),
            pltpu.SemaphoreType.DMA((4,)),
            pltpu.SemaphoreType.DMA((4,)),
        ],
        compiler_params=pltpu.CompilerParams(collective_id=0),
    )(A, B)


# device time: 33140 ns/iter; 1.5396x vs baseline; 1.5396x over previous
import jax
import jax.numpy as jnp
from jax import lax
from jax.experimental import pallas as pl
from jax.experimental.pallas import tpu as pltpu

N_DEV = 4
M = 1024
N = 1024
H = M // 2
Q = M // 4
C = N // 2

F32 = jnp.float32
BF16 = jnp.bfloat16


def kernel(A, B):
    def body(
        a_ref,
        b_ref,
        out_ref,
        acc_ref,
        h_send,
        h_recv,
        q_send,
        q_recv,
        gq_send,
        gq_recv,
        gh_send,
        gh_recv,
        send_sems,
        recv_sems,
    ):
        d = lax.axis_index("i")
        p1 = d ^ 1
        p2 = 3 - d

        keep0 = (d ^ (d >> 1)) & 1
        qi0 = keep0 * 2 + (d >> 1)
        qo0 = keep0 * 2 + (1 - (d >> 1))
        keep1 = d >> 1
        qi1 = keep1 * 2 + (d & 1)
        qo1 = keep1 * 2 + (1 - (d & 1))

        groups = [
            dict(g=0, pa=p1, pb=p2, keep=keep0, qi=qi0, qo=qo0, col=0),
            dict(g=1, pa=p2, pb=p1, keep=keep1, qi=qi1, qo=qo1, col=C),
        ]
        for gr in groups:
            gr["keep_r"] = gr["keep"] * H
            gr["send_r"] = (1 - gr["keep"]) * H
            gr["qi_r"] = gr["qi"] * Q
            gr["qo_r"] = gr["qo"] * Q

        barrier_sem = pltpu.get_barrier_semaphore()
        for nbr in [p1, p2]:
            pl.semaphore_signal(
                barrier_sem,
                inc=1,
                device_id=(nbr,),
                device_id_type=pl.DeviceIdType.MESH,
            )
        pl.semaphore_wait(barrier_sem, 2)

        def quad_mm(r, c):
            a = a_ref[pl.ds(r, H), :].astype(BF16)
            b = b_ref[:, pl.ds(c, C)].astype(BF16)
            return jnp.dot(a, b, preferred_element_type=F32)

        rdma1 = []
        for gr in groups:
            part = quad_mm(gr["send_r"], gr["col"])
            acc_ref[pl.ds(gr["send_r"], H), pl.ds(gr["col"], C)] = part
            h_send[gr["g"]] = part.astype(BF16)
            r = pltpu.make_async_remote_copy(
                src_ref=h_send.at[gr["g"]],
                dst_ref=h_recv.at[gr["g"]],
                send_sem=send_sems.at[gr["g"], 0],
                recv_sem=recv_sems.at[gr["g"], 0],
                device_id=(gr["pa"],),
                device_id_type=pl.DeviceIdType.MESH,
            )
            r.start()
            rdma1.append(r)
        for gr in groups:
            acc_ref[pl.ds(gr["keep_r"], H), pl.ds(gr["col"], C)] = quad_mm(
                gr["keep_r"], gr["col"]
            )
        rdma2 = []
        for gr, r1 in zip(groups, rdma1):
            r1.wait()
            cs = pl.ds(gr["col"], C)
            acc_ref[pl.ds(gr["keep_r"], H), cs] = (
                acc_ref[pl.ds(gr["keep_r"], H), cs]
                + h_recv[gr["g"]].astype(F32)
            )
            q_send[gr["g"]] = acc_ref[pl.ds(gr["qo_r"], Q), cs].astype(BF16)
            r = pltpu.make_async_remote_copy(
                src_ref=q_send.at[gr["g"]],
                dst_ref=q_recv.at[gr["g"]],
                send_sem=send_sems.at[gr["g"], 1],
                recv_sem=recv_sems.at[gr["g"], 1],
                device_id=(gr["pb"],),
                device_id_type=pl.DeviceIdType.MESH,
            )
            r.start()
            rdma2.append(r)
        rdma3 = []
        for gr, r2 in zip(groups, rdma2):
            r2.wait()
            cs = pl.ds(gr["col"], C)
            zq = acc_ref[pl.ds(gr["qi_r"], Q), cs] + q_recv[gr["g"]].astype(F32)
            gq = 0.5 * zq * (
                1.0 + jnp.tanh(0.7978845608 * (zq + 0.044715 * zq * zq * zq))
            )
            out_ref[pl.ds(gr["qi_r"], Q), cs] = gq
            gq_send[gr["g"]] = gq.astype(BF16)
            r = pltpu.make_async_remote_copy(
                src_ref=gq_send.at[gr["g"]],
                dst_ref=gq_recv.at[gr["g"]],
                send_sem=send_sems.at[gr["g"], 2],
                recv_sem=recv_sems.at[gr["g"], 2],
                device_id=(gr["pb"],),
                device_id_type=pl.DeviceIdType.MESH,
            )
            r.start()
            rdma3.append(r)
        rdma4 = []
        for gr, r3 in zip(groups, rdma3):
            r3.wait()
            cs = pl.ds(gr["col"], C)
            out_ref[pl.ds(gr["qo_r"], Q), cs] = gq_recv[gr["g"]].astype(F32)
            gh_send[gr["g"]] = out_ref[pl.ds(gr["keep_r"], H), cs].astype(BF16)
            r = pltpu.make_async_remote_copy(
                src_ref=gh_send.at[gr["g"]],
                dst_ref=gh_recv.at[gr["g"]],
                send_sem=send_sems.at[gr["g"], 3],
                recv_sem=recv_sems.at[gr["g"], 3],
                device_id=(gr["pa"],),
                device_id_type=pl.DeviceIdType.MESH,
            )
            r.start()
            rdma4.append(r)
        for gr, r4 in zip(groups, rdma4):
            r4.wait()
            out_ref[pl.ds(gr["send_r"], H), pl.ds(gr["col"], C)] = gh_recv[
                gr["g"]
            ].astype(F32)

    return pl.pallas_call(
        body,
        out_shape=jax.ShapeDtypeStruct((M, N), F32),
        in_specs=[
            pl.BlockSpec(memory_space=pltpu.VMEM),
            pl.BlockSpec(memory_space=pltpu.VMEM),
        ],
        out_specs=pl.BlockSpec(memory_space=pltpu.VMEM),
        scratch_shapes=[
            pltpu.VMEM((M, N), F32),
            pltpu.VMEM((2, H, C), BF16),
            pltpu.VMEM((2, H, C), BF16),
            pltpu.VMEM((2, Q, C), BF16),
            pltpu.VMEM((2, Q, C), BF16),
            pltpu.VMEM((2, Q, C), BF16),
            pltpu.VMEM((2, Q, C), BF16),
            pltpu.VMEM((2, H, C), BF16),
            pltpu.VMEM((2, H, C), BF16),
            pltpu.SemaphoreType.DMA((2, 4)),
            pltpu.SemaphoreType.DMA((2, 4)),
        ],
        compiler_params=pltpu.CompilerParams(collective_id=0),
    )(A, B)
